# Initial kernel scaffold; baseline (speedup 1.0000x reference)
import jax
import jax.numpy as jnp
from jax.experimental import pallas as pl


def kernel(nodes_in, inputs, W2, b2, W3, b3, W5, b5):
    raise NotImplementedError("write your pallas kernel here")


# speedup vs baseline: 10.8968x; 10.8968x over previous
import functools

import jax
import jax.numpy as jnp
from jax.experimental import pallas as pl

N = 8192
D = 128


def _graph_layer_body(nodes_ref, w2_ref, b2_ref, w3_ref, b3_ref,
                      w5_ref, b5_ref, out_ref):
    nodes = nodes_ref[:]
    h = jax.lax.dot_general(
        nodes, w3_ref[:], (((1,), (1,)), ((), ())),
        preferred_element_type=jnp.float32) + b3_ref[:]
    g2 = jax.lax.dot_general(
        nodes, w2_ref[:], (((1,), (1,)), ((), ())),
        preferred_element_type=jnp.float32) + b2_ref[:]
    s = jax.lax.dot_general(
        h, g2, (((0,), (0,)), ((), ())),
        preferred_element_type=jnp.float32)
    d = jnp.sum(h * h, axis=1, keepdims=True)
    msg = (jax.lax.dot_general(
        h, s, (((1,), (0,)), ((), ())),
        preferred_element_type=jnp.float32) - d * g2) * (1.0 / (N - 1))
    out_ref[:] = jax.lax.dot_general(
        msg, w5_ref[:], (((1,), (1,)), ((), ())),
        preferred_element_type=jnp.float32) + b5_ref[:] + nodes


@functools.partial(jax.jit, static_argnames=())
def kernel(nodes_in, inputs, W2, b2, W3, b3, W5, b5):
    del inputs
    b2r = b2.reshape(1, D)
    b3r = b3.reshape(1, D)
    b5r = b5.reshape(1, D)
    return pl.pallas_call(
        _graph_layer_body,
        out_shape=jax.ShapeDtypeStruct((N, D), jnp.float32),
    )(nodes_in, W2, b2r, W3, b3r, W5, b5r)
